# baseline (speedup 1.0000x reference)
import jax
import jax.numpy as jnp
import numpy as np
from jax import lax
from jax.experimental import pallas as pl
from jax.experimental.pallas import tpu as pltpu
from jax.experimental.pallas import tpu_sc as plsc

N = 10000
E = 320000
HID = 128
EED = 32
NL = 3
HF = 64
AW = 80
BN = 1000
CH = 128
NW = 32
NCHUNK = E // CH
NP = 10240
RPT = NP // 16


def _mm(a, b):
    return jnp.dot(a, b, precision=jax.lax.Precision.HIGHEST)


def _rb(x):
    return x.astype(jnp.bfloat16).astype(jnp.float32)


def _pre_body(h_ref, cond_ref, t_ref, frq_ref, w1a, w1b, w1s, w1c, b1,
              w2, b2, w3, b3, wd, ws, h_out, pd_out, ps_out):
    args = t_ref[...] * frq_ref[...]
    z = (_mm(_rb(h_ref[...]), _rb(w1a[...]))
         + _mm(_rb(cond_ref[...]), _rb(w1b[...]))
         + _mm(_rb(jnp.sin(args)), _rb(w1s[...]))
         + _mm(_rb(jnp.cos(args)), _rb(w1c[...])) + b1[...])
    h = jnp.maximum(z, 0.0)
    h = jnp.maximum(_mm(_rb(h), _rb(w2[...])) + b2[...], 0.0)
    h = _mm(_rb(h), _rb(w3[...])) + b3[...]
    h_out[...] = h
    hb = _rb(h)
    pd_out[...] = _mm(hb, _rb(wd[...]))
    ps_out[...] = _mm(hb, _rb(ws[...]))


def _mid_body(a0, a1, h_ref, x_ref, w2, b2v, wh1a, wh1b, bh1, wh2, bh2,
              wdn, wsn, h_out, x_out, pd_out, ps_out):
    Am = jnp.concatenate([a0[:, :HF], a1[:, :HF]], axis=1)
    tail = a0[:, HF:HF + 4] + a1[:, HF:HF + 4]
    deg = tail[:, 3:4]
    agg = _mm(Am, _rb(w2[...])) + deg * b2v[...]
    h = h_ref[...]
    hu = jnp.maximum(_mm(_rb(h), _rb(wh1a[...]))
                     + _mm(_rb(agg), _rb(wh1b[...])) + bh1[...], 0.0)
    hn = h + _mm(_rb(hu), _rb(wh2[...])) + bh2[...]
    h_out[...] = hn
    x_out[...] = x_ref[...] + tail / (deg + 1.0)
    hnb = _rb(hn)
    pd_out[...] = _mm(hnb, _rb(wdn[...]))
    ps_out[...] = _mm(hnb, _rb(wsn[...]))


def _fin_body(h_ref, x_ref, x0_ref, m_ref, whi, bhi, vh_out, vx_out):
    m = m_ref[...]
    vh_out[...] = m * (_mm(_rb(h_ref[...]), _rb(whi[...])) + bhi[...])
    vx_out[...] = m * (x_ref[...] - x0_ref[...])


def _row_spec(d):
    return pl.BlockSpec((BN, d), lambda i: (i, 0))


def _full_spec(r, d):
    return pl.BlockSpec((r, d), lambda i: (0, 0))


def _tc_pre(H_t, cond, t2, frq, weights):
    (w1a, w1b, w1s, w1c, b1, w2, b2, w3, b3, wd, ws) = weights
    grid = N // BN
    return pl.pallas_call(
        _pre_body,
        grid=(grid,),
        in_specs=[
            _row_spec(HID), _row_spec(HID), _row_spec(1), _full_spec(1, 64),
            _full_spec(HID, HID), _full_spec(HID, HID),
            _full_spec(64, HID), _full_spec(64, HID), _full_spec(1, HID),
            _full_spec(HID, HID), _full_spec(1, HID),
            _full_spec(HID, HID), _full_spec(1, HID),
            _full_spec(HID, HID), _full_spec(HID, HID),
        ],
        out_specs=[_row_spec(HID)] * 3,
        out_shape=[jax.ShapeDtypeStruct((N, HID), jnp.float32)] * 3,
    )(H_t, cond, t2, frq, w1a, w1b, w1s, w1c, b1, w2, b2, w3, b3, wd, ws)


def _tc_mid(a0, a1, h, x, weights):
    (w2, b2v, wh1a, wh1b, bh1, wh2, bh2, wdn, wsn) = weights
    grid = N // BN
    return pl.pallas_call(
        _mid_body,
        grid=(grid,),
        in_specs=[
            _row_spec(AW), _row_spec(AW), _row_spec(HID), _row_spec(4),
            _full_spec(HID, HID), _full_spec(1, HID),
            _full_spec(HID, HID), _full_spec(HID, HID), _full_spec(1, HID),
            _full_spec(HID, HID), _full_spec(1, HID),
            _full_spec(HID, HID), _full_spec(HID, HID),
        ],
        out_specs=[_row_spec(HID), _row_spec(4), _row_spec(HID),
                   _row_spec(HID)],
        out_shape=[
            jax.ShapeDtypeStruct((N, HID), jnp.float32),
            jax.ShapeDtypeStruct((N, 4), jnp.float32),
            jax.ShapeDtypeStruct((N, HID), jnp.float32),
            jax.ShapeDtypeStruct((N, HID), jnp.float32),
        ],
    )(a0, a1, h, x, w2, b2v, wh1a, wh1b, bh1, wh2, bh2, wdn, wsn)


def _tc_fin(h, x, x0, maskf, whi, bhi):
    grid = N // BN
    return pl.pallas_call(
        _fin_body,
        grid=(grid,),
        in_specs=[
            _row_spec(HID), _row_spec(4), _row_spec(4), _row_spec(1),
            _full_spec(HID, HID), _full_spec(1, HID),
        ],
        out_specs=[_row_spec(HID), _row_spec(4)],
        out_shape=[
            jax.ShapeDtypeStruct((N, HID), jnp.float32),
            jax.ShapeDtypeStruct((N, 4), jnp.float32),
        ],
    )(h, x, x0, maskf, whi, bhi)


def _sc_edge_kernel(pd_hbm, ps_hbm, x_hbm, src_hbm, dst_hbm, etf_hbm,
                    const_hbm, out_hbm, xbuf, srcbuf, dstbuf, etbuf,
                    svbuf, dvbuf, bufS, bufD, m1buf, geom, constbuf,
                    a_sh, sem1, sem2):
    cid = lax.axis_index("c")
    sid = lax.axis_index("s")

    pltpu.sync_copy(const_hbm.at[cid], constbuf)
    pltpu.sync_copy(x_hbm, xbuf)

    zz = jnp.zeros((16,), jnp.float32)

    def _zrow(r, carry):
        for j in range(AW // 16):
            m1buf[r, pl.ds(j * 16, 16)] = zz
        return carry

    lax.fori_loop(0, CH, _zrow, 0)
    base_n = sid * RPT
    for q in range(RPT // CH):
        pltpu.sync_copy(m1buf, a_sh.at[pl.ds(base_n + q * CH, CH)])
    plsc.subcore_barrier()

    bsv = [constbuf[0, pl.ds(j * 16, 16)] for j in range(4)]
    dlv = [constbuf[1, pl.ds(j * 16, 16)] for j in range(4)]
    w2v = [constbuf[2, pl.ds(j * 16, 16)] for j in range(4)]
    vvv = [constbuf[3, pl.ds(j * 16, 16)] for j in range(4)]
    crow = constbuf[4, pl.ds(0, 16)]
    c0 = crow[0]
    dflag = crow[1]
    io16 = lax.iota(jnp.int32, 16)

    def _rbf16(v):
        u = plsc.bitcast(v, jnp.uint32)
        r = (u + 0x7FFF + ((u >> 16) & 1)) & jnp.uint32(0xFFFF0000)
        return plsc.bitcast(r, jnp.float32)

    n_per = NCHUNK // 16
    extra = NCHUNK - n_per * 16
    lo = sid * n_per + jnp.minimum(sid, extra)
    hi = lo + n_per + jnp.where(sid < extra, 1, 0)

    def _chunk(c, carry):
        b = c * CH
        pltpu.sync_copy(src_hbm.at[pl.ds(b, CH)], srcbuf)
        pltpu.sync_copy(dst_hbm.at[pl.ds(b, CH)], dstbuf)
        pltpu.sync_copy(etf_hbm.at[pl.ds(b, CH)], etbuf)
        for g in range(8):
            sl = pl.ds(g * 16, 16)
            svbuf[sl] = srcbuf[sl] * 2 + cid
            dvbuf[sl] = dstbuf[sl] * 2 + cid
        g1 = pltpu.async_copy(ps_hbm.at[svbuf], bufS, sem1)
        g2 = pltpu.async_copy(pd_hbm.at[dvbuf], bufD, sem2)
        for g in range(8):
            s16 = srcbuf[pl.ds(g * 16, 16)] * 4
            d16 = dstbuf[pl.ds(g * 16, 16)] * 4
            d2 = jnp.zeros((16,), jnp.float32)
            for k in range(3):
                xs = plsc.load_gather(xbuf, [s16 + k])
                xd = plsc.load_gather(xbuf, [d16 + k])
                rk = xd - xs
                geom[pl.ds(k * CH + g * 16, 16)] = rk
                d2 = d2 + rk * rk
            geom[pl.ds(3 * CH + g * 16, 16)] = _rbf16(d2)
        g1.wait()
        g2.wait()

        for g in range(8):
            gsl = pl.ds(g * 16, 16)
            d2g = geom[pl.ds(3 * CH + g * 16, 16)]
            etg = etbuf[gsl]
            r0g = geom[pl.ds(0 * CH + g * 16, 16)]
            r1g = geom[pl.ds(1 * CH + g * 16, 16)]
            r2g = geom[pl.ds(2 * CH + g * 16, 16)]
            for e16 in range(16):
                e = g * 16 + e16
                d2s = d2g[e16]
                ets = etg[e16]
                acc = jnp.zeros((16,), jnp.float32)
                for j in range(4):
                    m1j = _rbf16(jnp.maximum(
                        bufD[e, pl.ds(j * 16, 16)]
                        + bufS[e, pl.ds(j * 16, 16)]
                        + d2s * w2v[j] + ets * dlv[j] + bsv[j], 0.0))
                    m1buf[e, pl.ds(j * 16, 16)] = m1j
                    acc = acc + m1j * vvv[j]
                coef = jnp.sum(acc) + c0
                m1buf[e, pl.ds(HF, 16)] = jnp.where(
                    io16 == 0, r0g[e16] * coef,
                    jnp.where(io16 == 1, r1g[e16] * coef,
                              jnp.where(io16 == 2, r2g[e16] * coef,
                                        jnp.where(io16 == 3, dflag, 0.0))))

        pltpu.sync_copy(m1buf, a_sh.at[dstbuf], add=True)
        return carry

    lax.fori_loop(lo, hi, _chunk, 0)
    plsc.subcore_barrier()
    for q in range(RPT // CH):
        sl = pl.ds(base_n + q * CH, CH)
        pltpu.sync_copy(a_sh.at[sl], out_hbm.at[cid, sl])


def _sc_edge(pd, ps, x4, src, dst, etf, consts):
    mesh = plsc.VectorSubcoreMesh(core_axis_name="c", subcore_axis_name="s")
    fn = pl.kernel(
        _sc_edge_kernel,
        mesh=mesh,
        compiler_params=pltpu.CompilerParams(
            needs_layout_passes=False, use_tc_tiling_on_sc=False),
        out_type=jax.ShapeDtypeStruct((2, NP, AW), jnp.float32),
        scratch_types=[
            pltpu.VMEM((N * 4,), jnp.float32),
            pltpu.VMEM((CH,), jnp.int32),
            pltpu.VMEM((CH,), jnp.int32),
            pltpu.VMEM((CH,), jnp.float32),
            pltpu.VMEM((CH,), jnp.int32),
            pltpu.VMEM((CH,), jnp.int32),
            pltpu.VMEM((CH, HF), jnp.float32),
            pltpu.VMEM((CH, HF), jnp.float32),
            pltpu.VMEM((CH, AW), jnp.float32),
            pltpu.VMEM((4 * CH,), jnp.float32),
            pltpu.VMEM((8, HF), jnp.float32),
            pltpu.VMEM_SHARED((NP, AW), jnp.float32),
            pltpu.SemaphoreType.DMA,
            pltpu.SemaphoreType.DMA,
        ],
    )
    return fn(pd, ps, x4, src, dst, etf, consts)


def kernel(H_t, X_t, cond_embedding, edges, edge_types, generate_mask,
           batch_ids, t, params):
    f32 = jnp.float32
    half = HID // 2
    frq = np.exp(-np.log(10000.0)
                 * np.arange(half, dtype=np.float32) / (half - 1))
    frq = jnp.asarray(frq)[None, :]

    mlp = params["input_mlp"]
    W1 = mlp[0]["w"]
    pre_w = (W1[:HID], W1[HID:2 * HID], W1[2 * HID:2 * HID + half],
             W1[2 * HID + half:], mlp[0]["b"][None, :],
             mlp[1]["w"], mlp[1]["b"][None, :],
             mlp[2]["w"], mlp[2]["b"][None, :])

    ee = params["edge_emb"]
    layer_consts = []
    layer_mid_w = []
    for lp in params["layers"]:
        We1, be1 = lp["phi_e1"]["w"], lp["phi_e1"]["b"]
        We2, be2 = lp["phi_e2"]["w"], lp["phi_e2"]["b"]
        wx = lp["phi_x"]["w"][:, 0]
        bx = lp["phi_x"]["b"][0]
        Wd, Ws = We1[:HID], We1[HID:2 * HID]
        w_d2 = We1[2 * HID]
        Wee = We1[2 * HID + 1:]
        eeb = jnp.asarray(ee, f32).astype(jnp.bfloat16).astype(f32)
        Weeb = Wee.astype(jnp.bfloat16).astype(f32)
        wxb = wx.astype(jnp.bfloat16).astype(f32)
        base = eeb[0] @ Weeb + be1
        delta = eeb[1] @ Weeb - eeb[0] @ Weeb
        v = We2.astype(jnp.bfloat16).astype(f32) @ wxb
        c0 = be2 @ wxb + bx
        w_d2 = w_d2.astype(jnp.bfloat16).astype(f32)
        consts = jnp.zeros((2, 8, HF), f32)
        for c in range(2):
            hs = slice(c * HF, (c + 1) * HF)
            consts = consts.at[c, 0].set(base[hs]).at[c, 1].set(delta[hs])
            consts = consts.at[c, 2].set(w_d2[hs]).at[c, 3].set(v[hs])
        consts = consts.at[0, 4, 0].set(c0)
        consts = consts.at[0, 4, 1].set(1.0)
        layer_consts.append((Wd, Ws, consts))
        Wh1 = lp["phi_h1"]["w"]
        layer_mid_w.append((We2, be2[None, :], Wh1[:HID], Wh1[HID:],
                            lp["phi_h1"]["b"][None, :], lp["phi_h2"]["w"],
                            lp["phi_h2"]["b"][None, :]))

    src = edges[0].astype(jnp.int32)
    dst = edges[1].astype(jnp.int32)
    etf = edge_types.astype(f32)
    x4 = jnp.pad(X_t, ((0, 0), (0, 1)))
    maskf = generate_mask.astype(f32)[:, None]
    t2 = t[:, None]

    wd0, ws0, _ = layer_consts[0]
    h, pd, ps = _tc_pre(H_t, cond_embedding, t2, frq,
                        pre_w + (wd0, ws0))

    full_mid = [layer_mid_w[l]
                + (layer_consts[(l + 1) % NL][0],
                   layer_consts[(l + 1) % NL][1]) for l in range(NL)]
    mw_stack = tuple(jnp.stack([full_mid[l][i] for l in range(NL)])
                     for i in range(9))
    consts_stack = jnp.stack([layer_consts[l][2] for l in range(NL)])

    def _layer(l, carry):
        h, x, pd, ps = carry
        consts = lax.dynamic_index_in_dim(consts_stack, l, keepdims=False)
        mw = tuple(lax.dynamic_index_in_dim(w, l, keepdims=False)
                   for w in mw_stack)
        parts = _sc_edge(pd.reshape(2 * N, HF), ps.reshape(2 * N, HF),
                         x.reshape(-1), src, dst, etf, consts)
        hn, xn, pdn, psn = _tc_mid(parts[0, :N], parts[1, :N], h, x, mw)
        return (hn, xn, pdn, psn)

    nl_opaque = NL + jnp.sum(t * 0.0).astype(jnp.int32)
    h, x, _, _ = lax.fori_loop(0, nl_opaque, _layer, (h, x4, pd, ps))

    whi = params["hidden2input"]["w"]
    bhi = params["hidden2input"]["b"][None, :]
    v_H, vx = _tc_fin(h, x, x4, maskf, whi, bhi)
    return v_H, vx[:, :3]


# speedup vs baseline: 1.2865x; 1.2865x over previous
import jax
import jax.numpy as jnp
import numpy as np
from jax import lax
from jax.experimental import pallas as pl
from jax.experimental.pallas import tpu as pltpu
from jax.experimental.pallas import tpu_sc as plsc

N = 10000
E = 320000
HID = 128
EED = 32
NL = 3
HF = 64
AW = 80
BN = 1000
CH = 128
NW = 32
NCHUNK = E // CH
NP = 10240
RPT = NP // 16


def _mm(a, b):
    return jnp.dot(a, b, precision=jax.lax.Precision.HIGHEST)


def _rb(x):
    return x.astype(jnp.bfloat16).astype(jnp.float32)


def _pre_body(h_ref, cond_ref, t_ref, frq_ref, w1a, w1b, w1s, w1c, b1,
              w2, b2, w3, b3, wd, ws, h_out, pd_out, ps_out):
    args = t_ref[...] * frq_ref[...]
    z = (_mm(_rb(h_ref[...]), _rb(w1a[...]))
         + _mm(_rb(cond_ref[...]), _rb(w1b[...]))
         + _mm(_rb(jnp.sin(args)), _rb(w1s[...]))
         + _mm(_rb(jnp.cos(args)), _rb(w1c[...])) + b1[...])
    h = jnp.maximum(z, 0.0)
    h = jnp.maximum(_mm(_rb(h), _rb(w2[...])) + b2[...], 0.0)
    h = _mm(_rb(h), _rb(w3[...])) + b3[...]
    h_out[...] = h
    hb = _rb(h)
    pd_out[...] = _mm(hb, _rb(wd[...]))
    ps_out[...] = _mm(hb, _rb(ws[...]))


def _mid_body(a0, a1, h_ref, x_ref, w2, b2v, wh1a, wh1b, bh1, wh2, bh2,
              wdn, wsn, h_out, x_out, pd_out, ps_out):
    Am = jnp.concatenate([a0[:, :HF], a1[:, :HF]], axis=1)
    tail = a0[:, HF:HF + 4] + a1[:, HF:HF + 4]
    deg = tail[:, 3:4]
    agg = _mm(Am, _rb(w2[...])) + deg * b2v[...]
    h = h_ref[...]
    hu = jnp.maximum(_mm(_rb(h), _rb(wh1a[...]))
                     + _mm(_rb(agg), _rb(wh1b[...])) + bh1[...], 0.0)
    hn = h + _mm(_rb(hu), _rb(wh2[...])) + bh2[...]
    h_out[...] = hn
    x_out[...] = x_ref[...] + tail / (deg + 1.0)
    hnb = _rb(hn)
    pd_out[...] = _mm(hnb, _rb(wdn[...]))
    ps_out[...] = _mm(hnb, _rb(wsn[...]))


def _fin_body(h_ref, x_ref, x0_ref, m_ref, whi, bhi, vh_out, vx_out):
    m = m_ref[...]
    vh_out[...] = m * (_mm(_rb(h_ref[...]), _rb(whi[...])) + bhi[...])
    vx_out[...] = m * (x_ref[...] - x0_ref[...])


def _row_spec(d):
    return pl.BlockSpec((BN, d), lambda i: (i, 0))


def _full_spec(r, d):
    return pl.BlockSpec((r, d), lambda i: (0, 0))


def _tc_pre(H_t, cond, t2, frq, weights):
    (w1a, w1b, w1s, w1c, b1, w2, b2, w3, b3, wd, ws) = weights
    grid = N // BN
    return pl.pallas_call(
        _pre_body,
        grid=(grid,),
        in_specs=[
            _row_spec(HID), _row_spec(HID), _row_spec(1), _full_spec(1, 64),
            _full_spec(HID, HID), _full_spec(HID, HID),
            _full_spec(64, HID), _full_spec(64, HID), _full_spec(1, HID),
            _full_spec(HID, HID), _full_spec(1, HID),
            _full_spec(HID, HID), _full_spec(1, HID),
            _full_spec(HID, HID), _full_spec(HID, HID),
        ],
        out_specs=[_row_spec(HID)] * 3,
        out_shape=[jax.ShapeDtypeStruct((N, HID), jnp.float32)] * 3,
    )(H_t, cond, t2, frq, w1a, w1b, w1s, w1c, b1, w2, b2, w3, b3, wd, ws)


def _tc_mid(a0, a1, h, x, weights):
    (w2, b2v, wh1a, wh1b, bh1, wh2, bh2, wdn, wsn) = weights
    grid = N // BN
    return pl.pallas_call(
        _mid_body,
        grid=(grid,),
        in_specs=[
            _row_spec(AW), _row_spec(AW), _row_spec(HID), _row_spec(4),
            _full_spec(HID, HID), _full_spec(1, HID),
            _full_spec(HID, HID), _full_spec(HID, HID), _full_spec(1, HID),
            _full_spec(HID, HID), _full_spec(1, HID),
            _full_spec(HID, HID), _full_spec(HID, HID),
        ],
        out_specs=[_row_spec(HID), _row_spec(4), _row_spec(HID),
                   _row_spec(HID)],
        out_shape=[
            jax.ShapeDtypeStruct((N, HID), jnp.float32),
            jax.ShapeDtypeStruct((N, 4), jnp.float32),
            jax.ShapeDtypeStruct((N, HID), jnp.float32),
            jax.ShapeDtypeStruct((N, HID), jnp.float32),
        ],
    )(a0, a1, h, x, w2, b2v, wh1a, wh1b, bh1, wh2, bh2, wdn, wsn)


def _tc_fin(h, x, x0, maskf, whi, bhi):
    grid = N // BN
    return pl.pallas_call(
        _fin_body,
        grid=(grid,),
        in_specs=[
            _row_spec(HID), _row_spec(4), _row_spec(4), _row_spec(1),
            _full_spec(HID, HID), _full_spec(1, HID),
        ],
        out_specs=[_row_spec(HID), _row_spec(4)],
        out_shape=[
            jax.ShapeDtypeStruct((N, HID), jnp.float32),
            jax.ShapeDtypeStruct((N, 4), jnp.float32),
        ],
    )(h, x, x0, maskf, whi, bhi)


def _sc_edge_kernel(pd_hbm, ps_hbm, x_hbm, src_hbm, dst_hbm, etf_hbm,
                    const_hbm, out_hbm, xbuf, srcbuf, dstbuf, etbuf,
                    svbuf, dvbuf, bufS, bufD, m1buf, geom, constbuf,
                    a_sh, sem1, sem2):
    cid = lax.axis_index("c")
    sid = lax.axis_index("s")

    pltpu.sync_copy(const_hbm.at[cid], constbuf)
    pltpu.sync_copy(x_hbm, xbuf)

    zz = jnp.zeros((16,), jnp.float32)

    def _zrow(r, carry):
        for j in range(AW // 16):
            m1buf[r, pl.ds(j * 16, 16)] = zz
        return carry

    lax.fori_loop(0, CH, _zrow, 0)
    base_n = sid * RPT
    for q in range(RPT // CH):
        pltpu.sync_copy(m1buf, a_sh.at[pl.ds(base_n + q * CH, CH)])
    plsc.subcore_barrier()

    bsv = [constbuf[0, pl.ds(j * 16, 16)] for j in range(4)]
    dlv = [constbuf[1, pl.ds(j * 16, 16)] for j in range(4)]
    w2v = [constbuf[2, pl.ds(j * 16, 16)] for j in range(4)]
    vvv = [constbuf[3, pl.ds(j * 16, 16)] for j in range(4)]
    crow = constbuf[4, pl.ds(0, 16)]
    c0 = crow[0]
    dflag = crow[1]
    io16 = lax.iota(jnp.int32, 16)

    def _rbf16(v):
        u = plsc.bitcast(v, jnp.uint32)
        r = (u + 0x8000) & jnp.uint32(0xFFFF0000)
        return plsc.bitcast(r, jnp.float32)

    n_per = NCHUNK // 16
    extra = NCHUNK - n_per * 16
    lo = sid * n_per + jnp.minimum(sid, extra)
    hi = lo + n_per + jnp.where(sid < extra, 1, 0)

    def _chunk(c, carry):
        b = c * CH
        pltpu.sync_copy(src_hbm.at[pl.ds(b, CH)], srcbuf)
        pltpu.sync_copy(dst_hbm.at[pl.ds(b, CH)], dstbuf)
        pltpu.sync_copy(etf_hbm.at[pl.ds(b, CH)], etbuf)
        for g in range(8):
            sl = pl.ds(g * 16, 16)
            svbuf[sl] = srcbuf[sl] * 2 + cid
            dvbuf[sl] = dstbuf[sl] * 2 + cid
        g1 = pltpu.async_copy(ps_hbm.at[svbuf], bufS, sem1)
        g2 = pltpu.async_copy(pd_hbm.at[dvbuf], bufD, sem2)
        for g in range(8):
            s16 = srcbuf[pl.ds(g * 16, 16)] * 4
            d16 = dstbuf[pl.ds(g * 16, 16)] * 4
            d2 = jnp.zeros((16,), jnp.float32)
            for k in range(3):
                xs = plsc.load_gather(xbuf, [s16 + k])
                xd = plsc.load_gather(xbuf, [d16 + k])
                rk = xd - xs
                geom[pl.ds(k * CH + g * 16, 16)] = rk
                d2 = d2 + rk * rk
            geom[pl.ds(3 * CH + g * 16, 16)] = _rbf16(d2)
        g1.wait()
        g2.wait()

        for g in range(8):
            gsl = pl.ds(g * 16, 16)
            d2g = geom[pl.ds(3 * CH + g * 16, 16)]
            etg = etbuf[gsl]
            r0g = geom[pl.ds(0 * CH + g * 16, 16)]
            r1g = geom[pl.ds(1 * CH + g * 16, 16)]
            r2g = geom[pl.ds(2 * CH + g * 16, 16)]
            for e16 in range(16):
                e = g * 16 + e16
                d2s = d2g[e16]
                ets = etg[e16]
                acc = jnp.zeros((16,), jnp.float32)
                for j in range(4):
                    m1j = _rbf16(jnp.maximum(
                        bufD[e, pl.ds(j * 16, 16)]
                        + bufS[e, pl.ds(j * 16, 16)]
                        + d2s * w2v[j] + ets * dlv[j] + bsv[j], 0.0))
                    m1buf[e, pl.ds(j * 16, 16)] = m1j
                    acc = acc + m1j * vvv[j]
                coef = jnp.sum(acc) + c0
                m1buf[e, pl.ds(HF, 16)] = jnp.where(
                    io16 == 0, r0g[e16] * coef,
                    jnp.where(io16 == 1, r1g[e16] * coef,
                              jnp.where(io16 == 2, r2g[e16] * coef,
                                        jnp.where(io16 == 3, dflag, 0.0))))

        pltpu.sync_copy(m1buf, a_sh.at[dstbuf], add=True)
        return carry

    lax.fori_loop(lo, hi, _chunk, 0)
    plsc.subcore_barrier()
    for q in range(RPT // CH):
        sl = pl.ds(base_n + q * CH, CH)
        pltpu.sync_copy(a_sh.at[sl], out_hbm.at[cid, sl])


def _sc_edge(pd, ps, x4, src, dst, etf, consts):
    mesh = plsc.VectorSubcoreMesh(core_axis_name="c", subcore_axis_name="s")
    fn = pl.kernel(
        _sc_edge_kernel,
        mesh=mesh,
        compiler_params=pltpu.CompilerParams(
            needs_layout_passes=False, use_tc_tiling_on_sc=False),
        out_type=jax.ShapeDtypeStruct((2, NP, AW), jnp.float32),
        scratch_types=[
            pltpu.VMEM((N * 4,), jnp.float32),
            pltpu.VMEM((CH,), jnp.int32),
            pltpu.VMEM((CH,), jnp.int32),
            pltpu.VMEM((CH,), jnp.float32),
            pltpu.VMEM((CH,), jnp.int32),
            pltpu.VMEM((CH,), jnp.int32),
            pltpu.VMEM((CH, HF), jnp.float32),
            pltpu.VMEM((CH, HF), jnp.float32),
            pltpu.VMEM((CH, AW), jnp.float32),
            pltpu.VMEM((4 * CH,), jnp.float32),
            pltpu.VMEM((8, HF), jnp.float32),
            pltpu.VMEM_SHARED((NP, AW), jnp.float32),
            pltpu.SemaphoreType.DMA,
            pltpu.SemaphoreType.DMA,
        ],
    )
    return fn(pd, ps, x4, src, dst, etf, consts)


def kernel(H_t, X_t, cond_embedding, edges, edge_types, generate_mask,
           batch_ids, t, params):
    f32 = jnp.float32
    half = HID // 2
    frq = np.exp(-np.log(10000.0)
                 * np.arange(half, dtype=np.float32) / (half - 1))
    frq = jnp.asarray(frq)[None, :]

    mlp = params["input_mlp"]
    W1 = mlp[0]["w"]
    pre_w = (W1[:HID], W1[HID:2 * HID], W1[2 * HID:2 * HID + half],
             W1[2 * HID + half:], mlp[0]["b"][None, :],
             mlp[1]["w"], mlp[1]["b"][None, :],
             mlp[2]["w"], mlp[2]["b"][None, :])

    ee = params["edge_emb"]
    layer_consts = []
    layer_mid_w = []
    for lp in params["layers"]:
        We1, be1 = lp["phi_e1"]["w"], lp["phi_e1"]["b"]
        We2, be2 = lp["phi_e2"]["w"], lp["phi_e2"]["b"]
        wx = lp["phi_x"]["w"][:, 0]
        bx = lp["phi_x"]["b"][0]
        Wd, Ws = We1[:HID], We1[HID:2 * HID]
        w_d2 = We1[2 * HID]
        Wee = We1[2 * HID + 1:]
        eeb = jnp.asarray(ee, f32).astype(jnp.bfloat16).astype(f32)
        Weeb = Wee.astype(jnp.bfloat16).astype(f32)
        wxb = wx.astype(jnp.bfloat16).astype(f32)
        base = eeb[0] @ Weeb + be1
        delta = eeb[1] @ Weeb - eeb[0] @ Weeb
        v = We2.astype(jnp.bfloat16).astype(f32) @ wxb
        c0 = be2 @ wxb + bx
        w_d2 = w_d2.astype(jnp.bfloat16).astype(f32)
        consts = jnp.zeros((2, 8, HF), f32)
        for c in range(2):
            hs = slice(c * HF, (c + 1) * HF)
            consts = consts.at[c, 0].set(base[hs]).at[c, 1].set(delta[hs])
            consts = consts.at[c, 2].set(w_d2[hs]).at[c, 3].set(v[hs])
        consts = consts.at[0, 4, 0].set(c0)
        consts = consts.at[0, 4, 1].set(1.0)
        layer_consts.append((Wd, Ws, consts))
        Wh1 = lp["phi_h1"]["w"]
        layer_mid_w.append((We2, be2[None, :], Wh1[:HID], Wh1[HID:],
                            lp["phi_h1"]["b"][None, :], lp["phi_h2"]["w"],
                            lp["phi_h2"]["b"][None, :]))

    src = edges[0].astype(jnp.int32)
    dst = edges[1].astype(jnp.int32)
    etf = edge_types.astype(f32)
    x4 = jnp.pad(X_t, ((0, 0), (0, 1)))
    maskf = generate_mask.astype(f32)[:, None]
    t2 = t[:, None]

    wd0, ws0, _ = layer_consts[0]
    h, pd, ps = _tc_pre(H_t, cond_embedding, t2, frq,
                        pre_w + (wd0, ws0))

    full_mid = [layer_mid_w[l]
                + (layer_consts[(l + 1) % NL][0],
                   layer_consts[(l + 1) % NL][1]) for l in range(NL)]
    mw_stack = tuple(jnp.stack([full_mid[l][i] for l in range(NL)])
                     for i in range(9))
    consts_stack = jnp.stack([layer_consts[l][2] for l in range(NL)])

    def _layer(l, carry):
        h, x, pd, ps = carry
        consts = lax.dynamic_index_in_dim(consts_stack, l, keepdims=False)
        mw = tuple(lax.dynamic_index_in_dim(w, l, keepdims=False)
                   for w in mw_stack)
        parts = _sc_edge(pd.reshape(2 * N, HF), ps.reshape(2 * N, HF),
                         x.reshape(-1), src, dst, etf, consts)
        hn, xn, pdn, psn = _tc_mid(parts[0, :N], parts[1, :N], h, x, mw)
        return (hn, xn, pdn, psn)

    nl_opaque = NL + jnp.sum(t * 0.0).astype(jnp.int32)
    h, x, _, _ = lax.fori_loop(0, nl_opaque, _layer, (h, x4, pd, ps))

    whi = params["hidden2input"]["w"]
    bhi = params["hidden2input"]["b"][None, :]
    v_H, vx = _tc_fin(h, x, x4, maskf, whi, bhi)
    return v_H, vx[:, :3]
